# Initial kernel scaffold; baseline (speedup 1.0000x reference)
import jax
import jax.numpy as jnp
from jax import lax
from jax.experimental import pallas as pl
from jax.experimental.pallas import tpu as pltpu
from jax.experimental.pallas import tpu_sc as plsc

N = 10000
E = 320000
B = 64
D = 128
NC = 2
NS = 16
NW = NC * NS
DEGW = 16
CHUNK = 80
EDGES_PER_TILE = E // NW
CHUNKS_PER_TILE = EDGES_PER_TILE // CHUNK
ROWS_PER_SUB = N // NS


def _sc_scatter_body(ef, src, dst, zrows, zdeg, ones,
                     s_out, deg_out,
                     rows_v, sidx_v, didx_v, ones_v, s_sh, deg_sh):
    c = lax.axis_index("c")
    s = lax.axis_index("s")
    wid = c * NS + s

    r0 = s * ROWS_PER_SUB
    pltpu.sync_copy(zrows.at[pl.ds(r0, ROWS_PER_SUB)],
                    s_sh.at[pl.ds(r0, ROWS_PER_SUB)])
    pltpu.sync_copy(zdeg.at[pl.ds(r0, ROWS_PER_SUB)],
                    deg_sh.at[pl.ds(r0, ROWS_PER_SUB)])
    pltpu.sync_copy(ones, ones_v)
    plsc.subcore_barrier()

    ebase = wid * EDGES_PER_TILE

    def step(i, carry):
        off = ebase + i * CHUNK
        pltpu.sync_copy(src.at[pl.ds(off, CHUNK)], sidx_v)
        pltpu.sync_copy(dst.at[pl.ds(off, CHUNK)], didx_v)
        pltpu.sync_copy(ef.at[pl.ds(off, CHUNK)], rows_v)
        pltpu.sync_copy(rows_v, s_sh.at[sidx_v], add=True)
        pltpu.sync_copy(rows_v, s_sh.at[didx_v], add=True)
        pltpu.sync_copy(ones_v, deg_sh.at[sidx_v], add=True)
        pltpu.sync_copy(ones_v, deg_sh.at[didx_v], add=True)
        return carry

    lax.fori_loop(0, CHUNKS_PER_TILE, step, 0)
    plsc.subcore_barrier()

    pltpu.sync_copy(s_sh.at[pl.ds(r0, ROWS_PER_SUB)],
                    s_out.at[pl.ds(c * N + r0, ROWS_PER_SUB)])
    pltpu.sync_copy(deg_sh.at[pl.ds(r0, ROWS_PER_SUB)],
                    deg_out.at[pl.ds(c * N + r0, ROWS_PER_SUB)])


def _tc_tail_body(en_ref, sparts_ref, dparts_ref, w_ref, b_ref, out_ref):
    en = en_ref[...]
    s_sum = sparts_ref[0:N, :] + sparts_ref[N:2 * N, :]
    deg = dparts_ref[0:N, :] + dparts_ref[N:2 * N, :]
    u = jnp.dot(en, s_sum, preferred_element_type=jnp.float32)
    rs = jnp.dot(en, deg, preferred_element_type=jnp.float32)
    row_sum = rs[:, 0:1]
    inv = 1.0 / row_sum
    inv = jnp.where(jnp.isinf(inv), 0.0, inv)
    scale = inv * row_sum
    hw = jnp.dot(u, w_ref[...], preferred_element_type=jnp.float32)
    out_ref[...] = hw * inv + scale * b_ref[...]


def kernel(edge_nodes, edge_feats, src, dst, weight, bias):
    zrows = jnp.zeros((N, D), jnp.float32)
    zdeg = jnp.zeros((N, DEGW), jnp.float32)
    ones = jnp.ones((CHUNK, DEGW), jnp.float32)

    sc = pl.kernel(
        _sc_scatter_body,
        out_type=(jax.ShapeDtypeStruct((NC * N, D), jnp.float32),
                  jax.ShapeDtypeStruct((NC * N, DEGW), jnp.float32)),
        mesh=plsc.VectorSubcoreMesh(core_axis_name="c", subcore_axis_name="s",
                                    num_cores=NC, num_subcores=NS),
        scratch_types=[
            pltpu.VMEM((CHUNK, D), jnp.float32),
            pltpu.VMEM((CHUNK,), jnp.int32),
            pltpu.VMEM((CHUNK,), jnp.int32),
            pltpu.VMEM((CHUNK, DEGW), jnp.float32),
            pltpu.VMEM_SHARED((N, D), jnp.float32),
            pltpu.VMEM_SHARED((N, DEGW), jnp.float32),
        ],
    )
    s_parts, deg_parts = sc(edge_feats, src, dst, zrows, zdeg, ones)

    h = pl.pallas_call(
        _tc_tail_body,
        out_shape=jax.ShapeDtypeStruct((B, D), jnp.float32),
    )(edge_nodes, s_parts, deg_parts, weight, bias.reshape(1, D))
    return h


# speedup vs baseline: 2.0110x; 2.0110x over previous
import jax
import jax.numpy as jnp
from jax import lax
from jax.experimental import pallas as pl
from jax.experimental.pallas import tpu as pltpu
from jax.experimental.pallas import tpu_sc as plsc

N = 10000
E = 320000
B = 64
D = 128
NC = 2
NS = 16
NW = NC * NS
CHUNK = 80
EDGES_PER_TILE = E // NW
CHUNKS_PER_TILE = EDGES_PER_TILE // CHUNK
NPAD = 10240
ROWS_PER_SUB = NPAD // NS
LANES = 16


def _sc_scatter_body(ef, ent, src, dst, zrows,
                     s_out, rs_out,
                     rows_v, sidx_v, didx_v, sg_v, dg_v, rs_v, s_sh):
    c = lax.axis_index("c")
    s = lax.axis_index("s")
    wid = c * NS + s

    r0 = s * ROWS_PER_SUB
    pltpu.sync_copy(zrows, rows_v)
    for j in range(ROWS_PER_SUB // CHUNK):
        pltpu.sync_copy(rows_v, s_sh.at[pl.ds(r0 + j * CHUNK, CHUNK)])
    plsc.subcore_barrier()

    ebase = wid * EDGES_PER_TILE
    zero16 = jnp.zeros((LANES,), jnp.float32)

    def step(i, acc):
        off = ebase + i * CHUNK
        pltpu.sync_copy(src.at[pl.ds(off, CHUNK)], sidx_v)
        pltpu.sync_copy(dst.at[pl.ds(off, CHUNK)], didx_v)
        pltpu.sync_copy(ef.at[pl.ds(off, CHUNK)], rows_v)
        pltpu.sync_copy(rows_v, s_sh.at[sidx_v], add=True)
        pltpu.sync_copy(rows_v, s_sh.at[didx_v], add=True)
        pltpu.sync_copy(ent.at[sidx_v], sg_v)
        pltpu.sync_copy(ent.at[didx_v], dg_v)
        accs = list(acc)
        for r in range(CHUNK):
            for k in range(B // LANES):
                accs[k] = (accs[k]
                           + sg_v[r, pl.ds(LANES * k, LANES)]
                           + dg_v[r, pl.ds(LANES * k, LANES)])
        return tuple(accs)

    acc = lax.fori_loop(0, CHUNKS_PER_TILE, step,
                        (zero16,) * (B // LANES))
    plsc.subcore_barrier()

    for k in range(B // LANES):
        rs_v[pl.ds(LANES * k, LANES)] = acc[k]
    pltpu.sync_copy(rs_v, rs_out.at[pl.ds(wid * B, B)])

    o0 = c * NPAD + r0
    for j in range(ROWS_PER_SUB // CHUNK):
        pltpu.sync_copy(s_sh.at[pl.ds(r0 + j * CHUNK, CHUNK)], rows_v)
        pltpu.sync_copy(rows_v, s_out.at[pl.ds(o0 + j * CHUNK, CHUNK)])


def _tc_tail_body(en_ref, sparts_ref, rs_ref, w_ref, b_ref, out_ref):
    en = en_ref[...]
    s_sum = sparts_ref[0:N, :] + sparts_ref[NPAD:NPAD + N, :]
    u = jnp.dot(en, s_sum, preferred_element_type=jnp.float32)
    rsall = rs_ref[...]
    row_sum = lax.dot_general(rsall, jnp.ones((NW, 1), jnp.float32),
                              dimension_numbers=(((0,), (0,)), ((), ())),
                              preferred_element_type=jnp.float32)
    inv = 1.0 / row_sum
    inv = jnp.where(jnp.isinf(inv), 0.0, inv)
    scale = inv * row_sum
    hw = jnp.dot(u, w_ref[...], preferred_element_type=jnp.float32)
    out_ref[...] = hw * inv + scale * b_ref[...]


def kernel(edge_nodes, edge_feats, src, dst, weight, bias):
    zrows = jnp.zeros((CHUNK, D), jnp.float32)
    ent = jnp.concatenate(
        [jnp.transpose(edge_nodes), jnp.zeros((N, D - B), jnp.float32)],
        axis=1)

    sc = pl.kernel(
        _sc_scatter_body,
        out_type=(jax.ShapeDtypeStruct((NC * NPAD, D), jnp.float32),
                  jax.ShapeDtypeStruct((NW * B,), jnp.float32)),
        mesh=plsc.VectorSubcoreMesh(core_axis_name="c", subcore_axis_name="s",
                                    num_cores=NC, num_subcores=NS),
        scratch_types=[
            pltpu.VMEM((CHUNK, D), jnp.float32),
            pltpu.VMEM((CHUNK,), jnp.int32),
            pltpu.VMEM((CHUNK,), jnp.int32),
            pltpu.VMEM((CHUNK, D), jnp.float32),
            pltpu.VMEM((CHUNK, D), jnp.float32),
            pltpu.VMEM((B,), jnp.float32),
            pltpu.VMEM_SHARED((NPAD, D), jnp.float32),
        ],
    )
    s_parts, rs_parts = sc(edge_feats, ent, src, dst, zrows)

    h = pl.pallas_call(
        _tc_tail_body,
        out_shape=jax.ShapeDtypeStruct((B, D), jnp.float32),
    )(edge_nodes, s_parts, rs_parts.reshape(NW, B), weight,
      bias.reshape(1, D))
    return h


# speedup vs baseline: 2.0625x; 1.0256x over previous
import jax
import jax.numpy as jnp
from jax import lax
from jax.experimental import pallas as pl
from jax.experimental.pallas import tpu as pltpu
from jax.experimental.pallas import tpu_sc as plsc

N = 10000
E = 320000
B = 64
D = 128
NC = 2
NS = 16
NW = NC * NS
BIGCHUNK = 128
EDGES_PER_TILE = E // NW
FULL_CHUNKS = EDGES_PER_TILE // BIGCHUNK
TAIL = EDGES_PER_TILE - FULL_CHUNKS * BIGCHUNK
NPAD = 10240
ROWS_PER_SUB = NPAD // NS
LANES = 16


def _sc_scatter_body(ef, ent, src, dst, zrows,
                     s_out, rs_out,
                     rows_v, sidx_v, didx_v, tidx_v, tdidx_v,
                     sg_v, rs_v, s_sh):
    c = lax.axis_index("c")
    s = lax.axis_index("s")
    wid = c * NS + s

    r0 = s * ROWS_PER_SUB
    pltpu.sync_copy(zrows, rows_v)
    for j in range(ROWS_PER_SUB // BIGCHUNK):
        pltpu.sync_copy(rows_v, s_sh.at[pl.ds(r0 + j * BIGCHUNK, BIGCHUNK)])
    plsc.subcore_barrier()

    ebase = wid * EDGES_PER_TILE
    zero16 = jnp.zeros((LANES,), jnp.float32)

    def accumulate(acc, nrows):
        accs = list(acc)
        for r in range(nrows):
            for k in range(B // LANES):
                accs[k] = accs[k] + sg_v[r, pl.ds(LANES * k, LANES)]
        return tuple(accs)

    def step(i, acc):
        off = ebase + i * BIGCHUNK
        pltpu.sync_copy(src.at[pl.ds(off, BIGCHUNK)], sidx_v)
        pltpu.sync_copy(dst.at[pl.ds(off, BIGCHUNK)], didx_v)
        pltpu.sync_copy(ef.at[pl.ds(off, BIGCHUNK)], rows_v)
        pltpu.sync_copy(rows_v, s_sh.at[sidx_v], add=True)
        pltpu.sync_copy(rows_v, s_sh.at[didx_v], add=True)
        pltpu.sync_copy(ent.at[sidx_v], sg_v)
        acc = accumulate(acc, BIGCHUNK)
        pltpu.sync_copy(ent.at[didx_v], sg_v)
        return accumulate(acc, BIGCHUNK)

    acc = lax.fori_loop(0, FULL_CHUNKS, step,
                        (zero16,) * (B // LANES))

    toff = ebase + FULL_CHUNKS * BIGCHUNK
    pltpu.sync_copy(src.at[pl.ds(toff, TAIL)], tidx_v)
    pltpu.sync_copy(dst.at[pl.ds(toff, TAIL)], tdidx_v)
    pltpu.sync_copy(ef.at[pl.ds(toff, TAIL)], rows_v.at[pl.ds(0, TAIL)])
    pltpu.sync_copy(rows_v.at[pl.ds(0, TAIL)], s_sh.at[tidx_v], add=True)
    pltpu.sync_copy(rows_v.at[pl.ds(0, TAIL)], s_sh.at[tdidx_v], add=True)
    pltpu.sync_copy(ent.at[tidx_v], sg_v.at[pl.ds(0, TAIL)])
    acc = accumulate(acc, TAIL)
    pltpu.sync_copy(ent.at[tdidx_v], sg_v.at[pl.ds(0, TAIL)])
    acc = accumulate(acc, TAIL)
    plsc.subcore_barrier()

    for k in range(B // LANES):
        rs_v[pl.ds(LANES * k, LANES)] = acc[k]
    pltpu.sync_copy(rs_v, rs_out.at[pl.ds(wid * B, B)])

    o0 = c * NPAD + r0
    for j in range(ROWS_PER_SUB // BIGCHUNK):
        pltpu.sync_copy(s_sh.at[pl.ds(r0 + j * BIGCHUNK, BIGCHUNK)], rows_v)
        pltpu.sync_copy(rows_v,
                        s_out.at[pl.ds(o0 + j * BIGCHUNK, BIGCHUNK)])


def _tc_tail_body(en_ref, sparts_ref, rs_ref, w_ref, b_ref, out_ref):
    en = en_ref[...]
    s_sum = sparts_ref[0:N, :] + sparts_ref[NPAD:NPAD + N, :]
    u = jnp.dot(en, s_sum, preferred_element_type=jnp.float32)
    rsall = rs_ref[...]
    row_sum = lax.dot_general(rsall, jnp.ones((NW, 1), jnp.float32),
                              dimension_numbers=(((0,), (0,)), ((), ())),
                              preferred_element_type=jnp.float32)
    inv = 1.0 / row_sum
    inv = jnp.where(jnp.isinf(inv), 0.0, inv)
    scale = inv * row_sum
    hw = jnp.dot(u, w_ref[...], preferred_element_type=jnp.float32)
    out_ref[...] = hw * inv + scale * b_ref[...]


def kernel(edge_nodes, edge_feats, src, dst, weight, bias):
    zrows = jnp.zeros((BIGCHUNK, D), jnp.float32)
    ent = jnp.concatenate(
        [jnp.transpose(edge_nodes), jnp.zeros((N, D - B), jnp.float32)],
        axis=1)

    sc = pl.kernel(
        _sc_scatter_body,
        out_type=(jax.ShapeDtypeStruct((NC * NPAD, D), jnp.float32),
                  jax.ShapeDtypeStruct((NW * B,), jnp.float32)),
        mesh=plsc.VectorSubcoreMesh(core_axis_name="c", subcore_axis_name="s",
                                    num_cores=NC, num_subcores=NS),
        scratch_types=[
            pltpu.VMEM((BIGCHUNK, D), jnp.float32),
            pltpu.VMEM((BIGCHUNK,), jnp.int32),
            pltpu.VMEM((BIGCHUNK,), jnp.int32),
            pltpu.VMEM((TAIL,), jnp.int32),
            pltpu.VMEM((TAIL,), jnp.int32),
            pltpu.VMEM((BIGCHUNK, D), jnp.float32),
            pltpu.VMEM((B,), jnp.float32),
            pltpu.VMEM_SHARED((NPAD, D), jnp.float32),
        ],
    )
    s_parts, rs_parts = sc(edge_feats, ent, src, dst, zrows)

    h = pl.pallas_call(
        _tc_tail_body,
        out_shape=jax.ShapeDtypeStruct((B, D), jnp.float32),
    )(edge_nodes, s_parts, rs_parts.reshape(NW, B), weight,
      bias.reshape(1, D))
    return h


# speedup vs baseline: 2.2218x; 1.0772x over previous
import jax
import jax.numpy as jnp
from jax import lax
from jax.experimental import pallas as pl
from jax.experimental.pallas import tpu as pltpu
from jax.experimental.pallas import tpu_sc as plsc

N = 10000
E = 320000
B = 64
D = 128
NC = 2
NS = 16
NW = NC * NS
BIGCHUNK = 128
EDGES_PER_TILE = E // NW
FULL_CHUNKS = EDGES_PER_TILE // BIGCHUNK
TAIL = EDGES_PER_TILE - FULL_CHUNKS * BIGCHUNK
NPAD = 10112
ROWS_PER_SUB = NPAD // NS
SUB_SLICES = [(0, 128), (128, 128), (256, 128), (384, 128), (512, 120)]
LANES = 16
GW = 64


def _sc_scatter_body(ef, ent, src, dst, zrows,
                     s_out, rs_out,
                     rows_v, sidx0_v, didx0_v,
                     tidx_v, tdidx_v, sg_v, dg_v, rs_v, s_sh, sem_d):
    c = lax.axis_index("c")
    s = lax.axis_index("s")
    wid = c * NS + s

    r0 = s * ROWS_PER_SUB
    pltpu.sync_copy(zrows, rows_v)
    for (joff, jlen) in SUB_SLICES:
        pltpu.sync_copy(rows_v.at[pl.ds(0, jlen)],
                        s_sh.at[pl.ds(r0 + joff, jlen)])
    plsc.subcore_barrier()

    ebase = wid * EDGES_PER_TILE
    zero16 = jnp.zeros((LANES,), jnp.float32)

    def accumulate(acc, nrows):
        accs = list(acc)
        for r in range(nrows):
            for k in range(B // LANES):
                accs[k] = (accs[k]
                           + sg_v[r, pl.ds(LANES * k, LANES)]
                           + dg_v[r, pl.ds(LANES * k, LANES)])
        return accs

    def chunk_ops(i, b, acc, prefetch):
        off = ebase + i * BIGCHUNK
        pltpu.sync_copy(src.at[pl.ds(off, BIGCHUNK)], sidx0_v)
        pltpu.sync_copy(dst.at[pl.ds(off, BIGCHUNK)], didx0_v)
        pltpu.sync_copy(ef.at[pl.ds(off, BIGCHUNK)], rows_v)
        pltpu.sync_copy(rows_v, s_sh.at[sidx0_v], add=True)
        pltpu.sync_copy(rows_v, s_sh.at[didx0_v], add=True)
        pltpu.sync_copy(ent.at[sidx0_v], sg_v)
        pltpu.sync_copy(ent.at[didx0_v], dg_v)
        return accumulate(acc, BIGCHUNK)

    acc = (zero16,) * (B // LANES)

    @pl.loop(0, FULL_CHUNKS, init_carry=tuple(acc))
    def acc(i, carry):
        return tuple(chunk_ops(i, 0, carry, False))

    toff = ebase + FULL_CHUNKS * BIGCHUNK
    pltpu.sync_copy(src.at[pl.ds(toff, TAIL)], tidx_v)
    pltpu.sync_copy(dst.at[pl.ds(toff, TAIL)], tdidx_v)
    pltpu.sync_copy(ef.at[pl.ds(toff, TAIL)], rows_v.at[pl.ds(0, TAIL)])
    pltpu.sync_copy(rows_v.at[pl.ds(0, TAIL)], s_sh.at[tidx_v], add=True)
    pltpu.sync_copy(rows_v.at[pl.ds(0, TAIL)], s_sh.at[tdidx_v], add=True)
    pltpu.sync_copy(ent.at[tidx_v], sg_v.at[pl.ds(0, TAIL)])
    pltpu.sync_copy(ent.at[tdidx_v], dg_v.at[pl.ds(0, TAIL)])
    acc = accumulate(acc, TAIL)
    plsc.subcore_barrier()

    for k in range(B // LANES):
        rs_v[pl.ds(LANES * k, LANES)] = acc[k]
    pltpu.async_copy(rs_v, rs_out.at[pl.ds(wid * B, B)], sem_d).wait()

    o0 = c * NPAD + r0
    for (joff, jlen) in SUB_SLICES:
        pltpu.sync_copy(s_sh.at[pl.ds(r0 + joff, jlen)],
                        rows_v.at[pl.ds(0, jlen)])
        pltpu.sync_copy(rows_v.at[pl.ds(0, jlen)],
                        s_out.at[pl.ds(o0 + joff, jlen)])


def _tc_tail_body(en_ref, sparts_ref, rs_ref, w_ref, b_ref, out_ref):
    en = en_ref[...]
    s_sum = sparts_ref[0:N, :] + sparts_ref[NPAD:NPAD + N, :]
    u = jnp.dot(en, s_sum, preferred_element_type=jnp.float32)
    rsall = rs_ref[...]
    row_sum = lax.dot_general(rsall, jnp.ones((NW, 1), jnp.float32),
                              dimension_numbers=(((0,), (0,)), ((), ())),
                              preferred_element_type=jnp.float32)
    inv = 1.0 / row_sum
    inv = jnp.where(jnp.isinf(inv), 0.0, inv)
    scale = inv * row_sum
    hw = jnp.dot(u, w_ref[...], preferred_element_type=jnp.float32)
    out_ref[...] = hw * inv + scale * b_ref[...]


def kernel(edge_nodes, edge_feats, src, dst, weight, bias):
    zrows = jnp.zeros((BIGCHUNK, D), jnp.float32)
    ent = jnp.transpose(edge_nodes)
    if GW > B:
        ent = jnp.concatenate([ent, jnp.zeros((N, GW - B), jnp.float32)],
                              axis=1)

    sc = pl.kernel(
        _sc_scatter_body,
        out_type=(jax.ShapeDtypeStruct((NC * NPAD, D), jnp.float32),
                  jax.ShapeDtypeStruct((NW * B,), jnp.float32)),
        mesh=plsc.VectorSubcoreMesh(core_axis_name="c", subcore_axis_name="s",
                                    num_cores=NC, num_subcores=NS),
        compiler_params=pltpu.CompilerParams(use_tc_tiling_on_sc=False),
        scratch_types=[
            pltpu.VMEM((BIGCHUNK, D), jnp.float32),
            pltpu.VMEM((BIGCHUNK,), jnp.int32),
            pltpu.VMEM((BIGCHUNK,), jnp.int32),
            pltpu.VMEM((TAIL,), jnp.int32),
            pltpu.VMEM((TAIL,), jnp.int32),
            pltpu.VMEM((BIGCHUNK, GW), jnp.float32),
            pltpu.VMEM((BIGCHUNK, GW), jnp.float32),
            pltpu.VMEM((B,), jnp.float32),
            pltpu.VMEM_SHARED((NPAD, D), jnp.float32),
            pltpu.SemaphoreType.DMA,
        ],
    )
    s_parts, rs_parts = sc(edge_feats, ent, src, dst, zrows)

    h = pl.pallas_call(
        _tc_tail_body,
        out_shape=jax.ShapeDtypeStruct((B, D), jnp.float32),
    )(edge_nodes, s_parts, rs_parts.reshape(NW, B), weight,
      bias.reshape(1, D))
    return h


# speedup vs baseline: 5.0504x; 2.2731x over previous
import jax
import jax.numpy as jnp
from jax import lax
from jax.experimental import pallas as pl
from jax.experimental.pallas import tpu as pltpu
from jax.experimental.pallas import tpu_sc as plsc

N = 10000
E = 320000
B = 64
D = 128
NC = 2
NS = 16
NW = NC * NS
BIGCHUNK = 128
EDGES_PER_TILE = E // NW
FULL_CHUNKS = EDGES_PER_TILE // BIGCHUNK
TAIL = EDGES_PER_TILE - FULL_CHUNKS * BIGCHUNK
NPAD = 10112
ROWS_PER_SUB = NPAD // NS
SUB_SLICES = [(0, 128), (128, 128), (256, 128), (384, 128), (512, 120)]
LANES = 16
DEGW = 16


def _sc_scatter_body(ef, src, dst, zrows, zdeg, ones,
                     s_out, deg_out,
                     rows_v, sidx0_v, didx0_v, tidx_v, tdidx_v,
                     ones_v, zdeg_v, s_sh, deg_sh):
    c = lax.axis_index("c")
    s = lax.axis_index("s")
    wid = c * NS + s

    r0 = s * ROWS_PER_SUB
    pltpu.sync_copy(zrows, rows_v)
    pltpu.sync_copy(zdeg, zdeg_v)
    pltpu.sync_copy(ones, ones_v)
    for (joff, jlen) in SUB_SLICES:
        pltpu.sync_copy(rows_v.at[pl.ds(0, jlen)],
                        s_sh.at[pl.ds(r0 + joff, jlen)])
        pltpu.sync_copy(zdeg_v.at[pl.ds(0, jlen)],
                        deg_sh.at[pl.ds(r0 + joff, jlen)])
    plsc.subcore_barrier()

    ebase = wid * EDGES_PER_TILE

    @pl.loop(0, FULL_CHUNKS)
    def _(i):
        off = ebase + i * BIGCHUNK
        pltpu.sync_copy(src.at[pl.ds(off, BIGCHUNK)], sidx0_v)
        pltpu.sync_copy(dst.at[pl.ds(off, BIGCHUNK)], didx0_v)
        pltpu.sync_copy(ef.at[pl.ds(off, BIGCHUNK)], rows_v)
        pltpu.sync_copy(rows_v, s_sh.at[sidx0_v], add=True)
        pltpu.sync_copy(rows_v, s_sh.at[didx0_v], add=True)
        pltpu.sync_copy(ones_v, deg_sh.at[sidx0_v], add=True)
        pltpu.sync_copy(ones_v, deg_sh.at[didx0_v], add=True)

    toff = ebase + FULL_CHUNKS * BIGCHUNK
    pltpu.sync_copy(src.at[pl.ds(toff, TAIL)], tidx_v)
    pltpu.sync_copy(dst.at[pl.ds(toff, TAIL)], tdidx_v)
    pltpu.sync_copy(ef.at[pl.ds(toff, TAIL)], rows_v.at[pl.ds(0, TAIL)])
    pltpu.sync_copy(rows_v.at[pl.ds(0, TAIL)], s_sh.at[tidx_v], add=True)
    pltpu.sync_copy(rows_v.at[pl.ds(0, TAIL)], s_sh.at[tdidx_v], add=True)
    pltpu.sync_copy(ones_v.at[pl.ds(0, TAIL)], deg_sh.at[tidx_v], add=True)
    pltpu.sync_copy(ones_v.at[pl.ds(0, TAIL)], deg_sh.at[tdidx_v], add=True)
    plsc.subcore_barrier()

    o0 = c * NPAD + r0
    for (joff, jlen) in SUB_SLICES:
        pltpu.sync_copy(s_sh.at[pl.ds(r0 + joff, jlen)],
                        rows_v.at[pl.ds(0, jlen)])
        pltpu.sync_copy(rows_v.at[pl.ds(0, jlen)],
                        s_out.at[pl.ds(o0 + joff, jlen)])
        pltpu.sync_copy(deg_sh.at[pl.ds(r0 + joff, jlen)],
                        zdeg_v.at[pl.ds(0, jlen)])
        pltpu.sync_copy(zdeg_v.at[pl.ds(0, jlen)],
                        deg_out.at[pl.ds(o0 + joff, jlen)])


def _tc_tail_body(en_ref, sparts_ref, dparts_ref, w_ref, b_ref, out_ref):
    en = en_ref[...]
    s_sum = sparts_ref[0:N, :] + sparts_ref[NPAD:NPAD + N, :]
    deg = dparts_ref[0:N, :] + dparts_ref[NPAD:NPAD + N, :]
    u = jnp.dot(en, s_sum, preferred_element_type=jnp.float32)
    rs = jnp.dot(en, deg, preferred_element_type=jnp.float32)
    row_sum = rs[:, 0:1]
    inv = 1.0 / row_sum
    inv = jnp.where(jnp.isinf(inv), 0.0, inv)
    scale = inv * row_sum
    hw = jnp.dot(u, w_ref[...], preferred_element_type=jnp.float32)
    out_ref[...] = hw * inv + scale * b_ref[...]


def kernel(edge_nodes, edge_feats, src, dst, weight, bias):
    zrows = jnp.zeros((BIGCHUNK, D), jnp.float32)
    zdeg = jnp.zeros((BIGCHUNK, DEGW), jnp.float32)
    ones = jnp.ones((BIGCHUNK, DEGW), jnp.float32)

    sc = pl.kernel(
        _sc_scatter_body,
        out_type=(jax.ShapeDtypeStruct((NC * NPAD, D), jnp.float32),
                  jax.ShapeDtypeStruct((NC * NPAD, DEGW), jnp.float32)),
        mesh=plsc.VectorSubcoreMesh(core_axis_name="c", subcore_axis_name="s",
                                    num_cores=NC, num_subcores=NS),
        compiler_params=pltpu.CompilerParams(use_tc_tiling_on_sc=False),
        scratch_types=[
            pltpu.VMEM((BIGCHUNK, D), jnp.float32),
            pltpu.VMEM((BIGCHUNK,), jnp.int32),
            pltpu.VMEM((BIGCHUNK,), jnp.int32),
            pltpu.VMEM((TAIL,), jnp.int32),
            pltpu.VMEM((TAIL,), jnp.int32),
            pltpu.VMEM((BIGCHUNK, DEGW), jnp.float32),
            pltpu.VMEM((BIGCHUNK, DEGW), jnp.float32),
            pltpu.VMEM_SHARED((NPAD, D), jnp.float32),
            pltpu.VMEM_SHARED((NPAD, DEGW), jnp.float32),
        ],
    )
    s_parts, deg_parts = sc(edge_feats, src, dst, zrows, zdeg, ones)

    h = pl.pallas_call(
        _tc_tail_body,
        out_shape=jax.ShapeDtypeStruct((B, D), jnp.float32),
    )(edge_nodes, s_parts, deg_parts, weight, bias.reshape(1, D))
    return h


# speedup vs baseline: 8.5904x; 1.7009x over previous
import jax
import jax.numpy as jnp
from jax import lax
from jax.experimental import pallas as pl
from jax.experimental.pallas import tpu as pltpu
from jax.experimental.pallas import tpu_sc as plsc

N = 10000
E = 320000
B = 64
D = 128
NC = 2
NS = 16
NW = NC * NS
BIGCHUNK = 128
EDGES_PER_TILE = E // NW
FULL_CHUNKS = EDGES_PER_TILE // BIGCHUNK
TAIL = EDGES_PER_TILE - FULL_CHUNKS * BIGCHUNK
NPAD = 10112
ROWS_PER_SUB = NPAD // NS
SUB_SLICES = [(0, 128), (128, 128), (256, 128), (384, 128), (512, 120)]
LANES = 16
DEGW = 8


def _sc_scatter_body(ef, src, dst, zrows, zdeg, ones,
                     s_out, deg_out,
                     rows0_v, rows1_v, sidx0_v, sidx1_v, didx0_v, didx1_v,
                     tidx_v, tdidx_v, ones_v, zdeg_v, s_sh, deg_sh,
                     sem_load, sem_scat):
    c = lax.axis_index("c")
    s = lax.axis_index("s")
    wid = c * NS + s

    r0 = s * ROWS_PER_SUB
    pltpu.sync_copy(zrows, rows0_v)
    pltpu.sync_copy(zdeg, zdeg_v)
    pltpu.sync_copy(ones, ones_v)
    for (joff, jlen) in SUB_SLICES:
        pltpu.sync_copy(rows0_v.at[pl.ds(0, jlen)],
                        s_sh.at[pl.ds(r0 + joff, jlen)])
        pltpu.sync_copy(zdeg_v.at[pl.ds(0, jlen)],
                        deg_sh.at[pl.ds(r0 + joff, jlen)])
    plsc.subcore_barrier()

    ebase = wid * EDGES_PER_TILE
    rows_b = (rows0_v, rows1_v)
    sidx_b = (sidx0_v, sidx1_v)
    didx_b = (didx0_v, didx1_v)

    pltpu.sync_copy(src.at[pl.ds(ebase, BIGCHUNK)], sidx0_v)
    pltpu.sync_copy(dst.at[pl.ds(ebase, BIGCHUNK)], didx0_v)
    pltpu.sync_copy(ef.at[pl.ds(ebase, BIGCHUNK)], rows0_v)

    load_d = [None, None]
    scat_d = [None, None]
    for i in range(FULL_CHUNKS):
        b = i % 2
        if load_d[b] is not None:
            for dsc in load_d[b]:
                dsc.wait()
            load_d[b] = None
        scat_d[b] = [
            pltpu.async_copy(rows_b[b], s_sh.at[sidx_b[b]],
                             sem_scat, add=True),
            pltpu.async_copy(rows_b[b], s_sh.at[didx_b[b]],
                             sem_scat, add=True),
            pltpu.async_copy(ones_v, deg_sh.at[sidx_b[b]],
                             sem_scat, add=True),
            pltpu.async_copy(ones_v, deg_sh.at[didx_b[b]],
                             sem_scat, add=True),
        ]
        if i + 1 < FULL_CHUNKS:
            nb = 1 - b
            if scat_d[nb] is not None:
                for dsc in scat_d[nb]:
                    dsc.wait()
                scat_d[nb] = None
            off = ebase + (i + 1) * BIGCHUNK
            load_d[nb] = [
                pltpu.async_copy(src.at[pl.ds(off, BIGCHUNK)],
                                 sidx_b[nb], sem_load),
                pltpu.async_copy(dst.at[pl.ds(off, BIGCHUNK)],
                                 didx_b[nb], sem_load),
                pltpu.async_copy(ef.at[pl.ds(off, BIGCHUNK)],
                                 rows_b[nb], sem_load),
            ]
    for slot in (0, 1):
        if scat_d[slot] is not None:
            for dsc in scat_d[slot]:
                dsc.wait()

    toff = ebase + FULL_CHUNKS * BIGCHUNK
    pltpu.sync_copy(src.at[pl.ds(toff, TAIL)], tidx_v)
    pltpu.sync_copy(dst.at[pl.ds(toff, TAIL)], tdidx_v)
    pltpu.sync_copy(ef.at[pl.ds(toff, TAIL)], rows0_v.at[pl.ds(0, TAIL)])
    pltpu.sync_copy(rows0_v.at[pl.ds(0, TAIL)], s_sh.at[tidx_v], add=True)
    pltpu.sync_copy(rows0_v.at[pl.ds(0, TAIL)], s_sh.at[tdidx_v], add=True)
    pltpu.sync_copy(ones_v.at[pl.ds(0, TAIL)], deg_sh.at[tidx_v], add=True)
    pltpu.sync_copy(ones_v.at[pl.ds(0, TAIL)], deg_sh.at[tdidx_v], add=True)
    plsc.subcore_barrier()

    o0 = c * NPAD + r0
    for (joff, jlen) in SUB_SLICES:
        pltpu.sync_copy(s_sh.at[pl.ds(r0 + joff, jlen)],
                        rows0_v.at[pl.ds(0, jlen)])
        pltpu.sync_copy(rows0_v.at[pl.ds(0, jlen)],
                        s_out.at[pl.ds(o0 + joff, jlen)])
        pltpu.sync_copy(deg_sh.at[pl.ds(r0 + joff, jlen)],
                        zdeg_v.at[pl.ds(0, jlen)])
        pltpu.sync_copy(zdeg_v.at[pl.ds(0, jlen)],
                        deg_out.at[pl.ds(o0 + joff, jlen)])


def _tc_tail_body(en_ref, sparts_ref, dparts_ref, w_ref, b_ref, out_ref):
    en = en_ref[...]
    s_sum = sparts_ref[0:N, :] + sparts_ref[NPAD:NPAD + N, :]
    deg = dparts_ref[0:N, :] + dparts_ref[NPAD:NPAD + N, :]
    u = jnp.dot(en, s_sum, preferred_element_type=jnp.float32)
    rs = jnp.dot(en, deg, preferred_element_type=jnp.float32)
    row_sum = rs[:, 0:1]
    inv = 1.0 / row_sum
    inv = jnp.where(jnp.isinf(inv), 0.0, inv)
    scale = inv * row_sum
    hw = jnp.dot(u, w_ref[...], preferred_element_type=jnp.float32)
    out_ref[...] = hw * inv + scale * b_ref[...]


def kernel(edge_nodes, edge_feats, src, dst, weight, bias):
    zrows = jnp.zeros((BIGCHUNK, D), jnp.float32)
    zdeg = jnp.zeros((BIGCHUNK, DEGW), jnp.float32)
    ones = jnp.ones((BIGCHUNK, DEGW), jnp.float32)

    sc = pl.kernel(
        _sc_scatter_body,
        out_type=(jax.ShapeDtypeStruct((NC * NPAD, D), jnp.float32),
                  jax.ShapeDtypeStruct((NC * NPAD, DEGW), jnp.float32)),
        mesh=plsc.VectorSubcoreMesh(core_axis_name="c", subcore_axis_name="s",
                                    num_cores=NC, num_subcores=NS),
        compiler_params=pltpu.CompilerParams(use_tc_tiling_on_sc=False),
        scratch_types=[
            pltpu.VMEM((BIGCHUNK, D), jnp.float32),
            pltpu.VMEM((BIGCHUNK, D), jnp.float32),
            pltpu.VMEM((BIGCHUNK,), jnp.int32),
            pltpu.VMEM((BIGCHUNK,), jnp.int32),
            pltpu.VMEM((BIGCHUNK,), jnp.int32),
            pltpu.VMEM((BIGCHUNK,), jnp.int32),
            pltpu.VMEM((TAIL,), jnp.int32),
            pltpu.VMEM((TAIL,), jnp.int32),
            pltpu.VMEM((BIGCHUNK, DEGW), jnp.float32),
            pltpu.VMEM((BIGCHUNK, DEGW), jnp.float32),
            pltpu.VMEM_SHARED((NPAD, D), jnp.float32),
            pltpu.VMEM_SHARED((NPAD, DEGW), jnp.float32),
            pltpu.SemaphoreType.DMA,
            pltpu.SemaphoreType.DMA,
        ],
    )
    s_parts, deg_parts = sc(edge_feats, src, dst, zrows, zdeg, ones)

    h = pl.pallas_call(
        _tc_tail_body,
        out_shape=jax.ShapeDtypeStruct((B, D), jnp.float32),
    )(edge_nodes, s_parts, deg_parts, weight, bias.reshape(1, D))
    return h
